# baseline (speedup 1.0000x reference)
import functools

import jax
import jax.numpy as jnp
from jax import lax
from jax.experimental import pallas as pl
from jax.experimental.pallas import tpu as pltpu
from jax.experimental.pallas import tpu_sc as plsc

N_NODES = 10000
N_EDGES = 320000
D = 128

NC = 2
NS = 16
NW = NC * NS
E_PER_W = N_EDGES // NW
K = 80
NCH = E_PER_W // K
RPT = 632
PIECE = 80


def _dense_body(x_ref, w1t_ref, b1_ref, w2t_ref, b2_ref, a1wt_ref, a1b_ref,
                a2wt_ref, a2b_ref, fj_ref, base_ref, a1_ref, a2_ref):
    x = x_ref[:]
    fi = jnp.maximum(
        jnp.dot(x, w1t_ref[:], preferred_element_type=jnp.float32) + b1_ref[:], 0.0)
    fj = jnp.maximum(
        jnp.dot(x, w2t_ref[:], preferred_element_type=jnp.float32) + b2_ref[:], 0.0)
    a1 = jnp.dot(fi, a1wt_ref[:], preferred_element_type=jnp.float32) + a1b_ref[0, 0]
    a2 = jnp.dot(fj, a2wt_ref[:], preferred_element_type=jnp.float32) + a2b_ref[0, 0]
    att_self = jax.nn.sigmoid(a1 + a2)
    fj_ref[:] = fj
    base_ref[:] = fi + att_self * fj
    a1_ref[:] = a1
    a2_ref[:] = a2


def _dense_stage(features, W1T, b1, W2T, b2, a1wT, a1b, a2wT, a2b):
    R = 1000
    grid = N_NODES // R
    full = lambda shp: pl.BlockSpec(shp, lambda i: (0, 0))
    return pl.pallas_call(
        _dense_body,
        grid=(grid,),
        in_specs=[
            pl.BlockSpec((R, D), lambda i: (i, 0)),
            full((D, D)), full((1, D)), full((D, D)), full((1, D)),
            full((D, 1)), full((1, 1)), full((D, 1)), full((1, 1)),
        ],
        out_specs=[
            pl.BlockSpec((R, D), lambda i: (i, 0)),
            pl.BlockSpec((R, D), lambda i: (i, 0)),
            pl.BlockSpec((R, 1), lambda i: (i, 0)),
            pl.BlockSpec((R, 1), lambda i: (i, 0)),
        ],
        out_shape=[
            jax.ShapeDtypeStruct((N_NODES, D), jnp.float32),
            jax.ShapeDtypeStruct((N_NODES, D), jnp.float32),
            jax.ShapeDtypeStruct((N_NODES, 1), jnp.float32),
            jax.ShapeDtypeStruct((N_NODES, 1), jnp.float32),
        ],
    )(features, W1T, b1, W2T, b2, a1wT, a1b, a2wT, a2b)


NSLOT = 5


def _edge_body(fj_hbm, a1_hbm, a2_hbm, idx_hbm, part_hbm,
               idx_c, a1c, a2c, raw0, raw1, scd0, scd1, att_v,
               a1_sh, a2_sh, acc,
               sem_idx, sem_g, sem_a1, sem_a2, sem_sc):
    c = lax.axis_index("c")
    s = lax.axis_index("s")
    wid = c * NS + s
    ebase = wid * E_PER_W
    base = s * RPT
    rows_total = jnp.where(s < NS - 1, RPT, N_NODES - (NS - 1) * RPT)
    n_pieces = (rows_total + PIECE - 1) // PIECE

    def _zrow(i, carry):
        z = jnp.zeros((16,), jnp.float32)
        for q in range(8):
            raw0[i, pl.ds(q * 16, 16)] = z
        return carry
    lax.fori_loop(0, PIECE, _zrow, 0)

    def _zpiece(k, carry):
        off = base + jnp.minimum(k * PIECE, rows_total - PIECE)
        pltpu.sync_copy(raw0, acc.at[pl.ds(off, PIECE)])
        return carry
    lax.fori_loop(0, n_pieces, _zpiece, 0)

    @pl.when(s == 0)
    def _():
        pltpu.sync_copy(a1_hbm, a1_sh)

    @pl.when(s == 1)
    def _():
        pltpu.sync_copy(a2_hbm, a2_sh)

    plsc.subcore_barrier()

    raws = (raw0, raw1)
    scds = (scd0, scd1)

    def idx_start(j):
        m = j % NSLOT
        pltpu.async_copy(idx_hbm.at[wid, j], idx_c.at[m], sem_idx.at[m])

    def fetch_start(j, b):
        m = j % NSLOT
        pltpu.make_async_copy(idx_hbm.at[wid, j], idx_c.at[m],
                              sem_idx.at[m]).wait()
        pltpu.async_copy(fj_hbm.at[idx_c.at[m, 1]], raws[b], sem_g.at[b])
        pltpu.async_copy(a1_sh.at[idx_c.at[m, 0]], a1c.at[b], sem_a1.at[b])
        pltpu.async_copy(a2_sh.at[idx_c.at[m, 1]], a2c.at[b], sem_a2.at[b])

    def proc(j, b, wait_prev):
        m = j % NSLOT
        raw = raws[b]
        scd = scds[b]
        pltpu.make_async_copy(fj_hbm.at[idx_c.at[m, 1]], raw,
                              sem_g.at[b]).wait()
        pltpu.make_async_copy(a1_sh.at[idx_c.at[m, 0]], a1c.at[b],
                              sem_a1.at[b]).wait()
        pltpu.make_async_copy(a2_sh.at[idx_c.at[m, 1]], a2c.at[b],
                              sem_a2.at[b]).wait()
        if wait_prev:
            m2 = (j - 2) % NSLOT
            pltpu.make_async_copy(scd, acc.at[idx_c.at[m2, 0]],
                                  sem_sc.at[b]).wait()

        for g in range(5):
            x = a1c[b, pl.ds(g * 16, 16)] + a2c[b, pl.ds(g * 16, 16)]
            att_v[pl.ds(g * 16, 16)] = 1.0 / (1.0 + jnp.exp(-x))

        @plsc.parallel_loop(0, K, 1, unroll=4)
        def _scale(e):
            idx = lax.broadcast_in_dim(e, (16,), ())
            a = plsc.load_gather(att_v, [idx])
            for q in range(8):
                scd[e, pl.ds(q * 16, 16)] = raw[e, pl.ds(q * 16, 16)] * a

        pltpu.async_copy(scd, acc.at[idx_c.at[m, 0]], sem_sc.at[b], add=True)

    idx_start(0)
    idx_start(1)
    idx_start(2)
    idx_start(3)
    fetch_start(0, 0)
    fetch_start(1, 1)
    proc(0, 0, False)
    fetch_start(2, 0)
    proc(1, 1, False)

    def _pair(t, carry):
        ja = 2 * t
        jb = ja + 1
        idx_start(ja + 2)
        fetch_start(jb, 1)
        proc(ja, 0, True)
        idx_start(jb + 2)
        fetch_start(jb + 1, 0)
        proc(jb, 1, True)
        return carry
    lax.fori_loop(1, 61, _pair, 0)

    idx_start(124)
    fetch_start(123, 1)
    proc(122, 0, True)
    fetch_start(124, 0)
    proc(123, 1, True)
    proc(124, 0, True)
    pltpu.make_async_copy(scds[1], acc.at[idx_c.at[123 % NSLOT, 0]],
                          sem_sc.at[1]).wait()
    pltpu.make_async_copy(scds[0], acc.at[idx_c.at[124 % NSLOT, 0]],
                          sem_sc.at[0]).wait()

    plsc.subcore_barrier()

    def _cpiece(k, carry):
        off = base + jnp.minimum(k * PIECE, rows_total - PIECE)
        pltpu.sync_copy(acc.at[pl.ds(off, PIECE)], raw0)
        pltpu.sync_copy(raw0, part_hbm.at[c].at[pl.ds(off, PIECE)])
        return carry
    lax.fori_loop(0, n_pieces, _cpiece, 0)


def _edge_stage(fj, a1, a2, idx2):
    mesh = plsc.VectorSubcoreMesh(
        core_axis_name="c", subcore_axis_name="s", num_cores=NC, num_subcores=NS)
    kfn = pl.kernel(
        _edge_body,
        out_type=jax.ShapeDtypeStruct((NC, N_NODES, D), jnp.float32),
        mesh=mesh,
        scratch_types=[
            pltpu.VMEM((NSLOT, 2, K), jnp.int32),
            pltpu.VMEM((2, K), jnp.float32),
            pltpu.VMEM((2, K), jnp.float32),
            pltpu.VMEM((K, D), jnp.float32),
            pltpu.VMEM((K, D), jnp.float32),
            pltpu.VMEM((K, D), jnp.float32),
            pltpu.VMEM((K, D), jnp.float32),
            pltpu.VMEM((K,), jnp.float32),
            pltpu.VMEM_SHARED((N_NODES,), jnp.float32),
            pltpu.VMEM_SHARED((N_NODES,), jnp.float32),
            pltpu.VMEM_SHARED((N_NODES, D), jnp.float32),
            pltpu.SemaphoreType.DMA((NSLOT,)),
            pltpu.SemaphoreType.DMA((2,)),
            pltpu.SemaphoreType.DMA((2,)),
            pltpu.SemaphoreType.DMA((2,)),
            pltpu.SemaphoreType.DMA((2,)),
        ],
        compiler_params=pltpu.CompilerParams(needs_layout_passes=False),
    )
    return kfn(fj, a1, a2, idx2)


def _combine_body(p0_ref, p1_ref, base_ref, out_ref):
    out_ref[:] = p0_ref[:] + p1_ref[:] + base_ref[:]


def _combine_stage(p0, p1, base):
    R = 1000
    return pl.pallas_call(
        _combine_body,
        grid=(N_NODES // R,),
        in_specs=[pl.BlockSpec((R, D), lambda i: (i, 0))] * 3,
        out_specs=pl.BlockSpec((R, D), lambda i: (i, 0)),
        out_shape=jax.ShapeDtypeStruct((N_NODES, D), jnp.float32),
    )(p0, p1, base)


def kernel(features, adj_indices, W1, b1, W2, b2, a1_w, a1_b, a2_w, a2_b):
    adj = adj_indices.astype(jnp.int32)
    src_r = adj[0].reshape(NW, NCH, K)
    dst_r = adj[1].reshape(NW, NCH, K)
    idx2 = jnp.stack([src_r, dst_r], axis=2)

    fj, base, a1, a2 = _dense_stage(
        features,
        W1.T, b1.reshape(1, D),
        W2.T, b2.reshape(1, D),
        a1_w.reshape(1, D).T, a1_b.reshape(1, 1),
        a2_w.reshape(1, D).T, a2_b.reshape(1, 1),
    )

    part = _edge_stage(fj, a1.reshape(N_NODES), a2.reshape(N_NODES), idx2)

    return _combine_stage(part[0], part[1], base)


# speedup vs baseline: 1.0781x; 1.0781x over previous
import functools

import jax
import jax.numpy as jnp
from jax import lax
from jax.experimental import pallas as pl
from jax.experimental.pallas import tpu as pltpu
from jax.experimental.pallas import tpu_sc as plsc

N_NODES = 10000
N_EDGES = 320000
D = 128

NC = 2
NS = 16
NW = NC * NS
E_PER_W = N_EDGES // NW
K = 80
NCH = E_PER_W // K
RPT = 632
PIECE = 80


def _dense_body(x_ref, w1t_ref, b1_ref, w2t_ref, b2_ref, a1wt_ref, a1b_ref,
                a2wt_ref, a2b_ref, fj_ref, base_ref, a1_ref, a2_ref):
    x = x_ref[:]
    fi = jnp.maximum(
        jnp.dot(x, w1t_ref[:], preferred_element_type=jnp.float32) + b1_ref[:], 0.0)
    fj = jnp.maximum(
        jnp.dot(x, w2t_ref[:], preferred_element_type=jnp.float32) + b2_ref[:], 0.0)
    a1 = jnp.dot(fi, a1wt_ref[:], preferred_element_type=jnp.float32) + a1b_ref[0, 0]
    a2 = jnp.dot(fj, a2wt_ref[:], preferred_element_type=jnp.float32) + a2b_ref[0, 0]
    att_self = jax.nn.sigmoid(a1 + a2)
    fj_ref[:] = fj
    base_ref[:] = fi + att_self * fj
    a1_ref[:] = a1
    a2_ref[:] = a2


def _dense_stage(features, W1T, b1, W2T, b2, a1wT, a1b, a2wT, a2b):
    R = 1000
    grid = N_NODES // R
    full = lambda shp: pl.BlockSpec(shp, lambda i: (0, 0))
    return pl.pallas_call(
        _dense_body,
        grid=(grid,),
        in_specs=[
            pl.BlockSpec((R, D), lambda i: (i, 0)),
            full((D, D)), full((1, D)), full((D, D)), full((1, D)),
            full((D, 1)), full((1, 1)), full((D, 1)), full((1, 1)),
        ],
        out_specs=[
            pl.BlockSpec((R, D), lambda i: (i, 0)),
            pl.BlockSpec((R, D), lambda i: (i, 0)),
            pl.BlockSpec((R, 1), lambda i: (i, 0)),
            pl.BlockSpec((R, 1), lambda i: (i, 0)),
        ],
        out_shape=[
            jax.ShapeDtypeStruct((N_NODES, D), jnp.float32),
            jax.ShapeDtypeStruct((N_NODES, D), jnp.float32),
            jax.ShapeDtypeStruct((N_NODES, 1), jnp.float32),
            jax.ShapeDtypeStruct((N_NODES, 1), jnp.float32),
        ],
    )(features, W1T, b1, W2T, b2, a1wT, a1b, a2wT, a2b)


NSLOT = 5


def _edge_body(fj_hbm, a1_hbm, a2_hbm, src_hbm, dst_hbm, part_hbm,
               src_c, dst_c, a1c, a2c, raw0, raw1, scd0, scd1, att_v,
               a1_sh, a2_sh, acc,
               sem_src, sem_dst, sem_g, sem_a1, sem_a2, sem_sc):
    c = lax.axis_index("c")
    s = lax.axis_index("s")
    wid = c * NS + s
    ebase = wid * E_PER_W
    base = s * RPT
    rows_total = jnp.where(s < NS - 1, RPT, N_NODES - (NS - 1) * RPT)
    n_pieces = (rows_total + PIECE - 1) // PIECE

    def _zrow(i, carry):
        z = jnp.zeros((16,), jnp.float32)
        for q in range(8):
            raw0[i, pl.ds(q * 16, 16)] = z
        return carry
    lax.fori_loop(0, PIECE, _zrow, 0)

    def _zpiece(k, carry):
        off = base + jnp.minimum(k * PIECE, rows_total - PIECE)
        pltpu.sync_copy(raw0, acc.at[pl.ds(off, PIECE)])
        return carry
    lax.fori_loop(0, n_pieces, _zpiece, 0)

    @pl.when(s == 0)
    def _():
        pltpu.sync_copy(a1_hbm, a1_sh)

    @pl.when(s == 1)
    def _():
        pltpu.sync_copy(a2_hbm, a2_sh)

    plsc.subcore_barrier()

    raws = (raw0, raw1)
    scds = (scd0, scd1)

    def idx_start(j):
        m = j % NSLOT
        pltpu.async_copy(src_hbm.at[pl.ds(ebase + j * K, K)], src_c.at[m],
                         sem_src.at[m])
        pltpu.async_copy(dst_hbm.at[pl.ds(ebase + j * K, K)], dst_c.at[m],
                         sem_dst.at[m])

    def fetch_start(j, b):
        m = j % NSLOT
        pltpu.make_async_copy(src_hbm.at[pl.ds(ebase + j * K, K)],
                              src_c.at[m], sem_src.at[m]).wait()
        pltpu.make_async_copy(dst_hbm.at[pl.ds(ebase + j * K, K)],
                              dst_c.at[m], sem_dst.at[m]).wait()
        pltpu.async_copy(fj_hbm.at[dst_c.at[m]], raws[b], sem_g.at[b])
        pltpu.async_copy(a1_sh.at[src_c.at[m]], a1c.at[b], sem_a1.at[b])
        pltpu.async_copy(a2_sh.at[dst_c.at[m]], a2c.at[b], sem_a2.at[b])

    def proc(j, b, wait_prev):
        m = j % NSLOT
        raw = raws[b]
        scd = scds[b]
        pltpu.make_async_copy(fj_hbm.at[dst_c.at[m]], raw, sem_g.at[b]).wait()
        pltpu.make_async_copy(a1_sh.at[src_c.at[m]], a1c.at[b],
                              sem_a1.at[b]).wait()
        pltpu.make_async_copy(a2_sh.at[dst_c.at[m]], a2c.at[b],
                              sem_a2.at[b]).wait()
        if wait_prev:
            m2 = (j - 2) % NSLOT
            pltpu.make_async_copy(scd, acc.at[src_c.at[m2]],
                                  sem_sc.at[b]).wait()

        for g in range(5):
            x = a1c[b, pl.ds(g * 16, 16)] + a2c[b, pl.ds(g * 16, 16)]
            att_v[pl.ds(g * 16, 16)] = 1.0 / (1.0 + jnp.exp(-x))

        @plsc.parallel_loop(0, K, 1, unroll=4)
        def _scale(e):
            idx = lax.broadcast_in_dim(e, (16,), ())
            a = plsc.load_gather(att_v, [idx])
            for q in range(8):
                scd[e, pl.ds(q * 16, 16)] = raw[e, pl.ds(q * 16, 16)] * a

        pltpu.async_copy(scd, acc.at[src_c.at[m]], sem_sc.at[b], add=True)

    idx_start(0)
    idx_start(1)
    idx_start(2)
    idx_start(3)
    fetch_start(0, 0)
    fetch_start(1, 1)
    proc(0, 0, False)
    fetch_start(2, 0)
    proc(1, 1, False)

    def _pair(t, carry):
        ja = 2 * t
        jb = ja + 1
        idx_start(ja + 2)
        fetch_start(jb, 1)
        proc(ja, 0, True)
        idx_start(jb + 2)
        fetch_start(jb + 1, 0)
        proc(jb, 1, True)
        return carry
    lax.fori_loop(1, 61, _pair, 0)

    idx_start(124)
    fetch_start(123, 1)
    proc(122, 0, True)
    fetch_start(124, 0)
    proc(123, 1, True)
    proc(124, 0, True)
    pltpu.make_async_copy(scds[1], acc.at[src_c.at[123 % NSLOT]],
                          sem_sc.at[1]).wait()
    pltpu.make_async_copy(scds[0], acc.at[src_c.at[124 % NSLOT]],
                          sem_sc.at[0]).wait()

    plsc.subcore_barrier()

    def _cpiece(k, carry):
        off = base + jnp.minimum(k * PIECE, rows_total - PIECE)
        pltpu.sync_copy(acc.at[pl.ds(off, PIECE)], raw0)
        pltpu.sync_copy(raw0, part_hbm.at[c].at[pl.ds(off, PIECE)])
        return carry
    lax.fori_loop(0, n_pieces, _cpiece, 0)


def _edge_stage(fj, a1, a2, src_r, dst_r):
    mesh = plsc.VectorSubcoreMesh(
        core_axis_name="c", subcore_axis_name="s", num_cores=NC, num_subcores=NS)
    kfn = pl.kernel(
        _edge_body,
        out_type=jax.ShapeDtypeStruct((NC, N_NODES, D), jnp.float32),
        mesh=mesh,
        scratch_types=[
            pltpu.VMEM((NSLOT, K), jnp.int32),
            pltpu.VMEM((NSLOT, K), jnp.int32),
            pltpu.VMEM((2, K), jnp.float32),
            pltpu.VMEM((2, K), jnp.float32),
            pltpu.VMEM((K, D), jnp.float32),
            pltpu.VMEM((K, D), jnp.float32),
            pltpu.VMEM((K, D), jnp.float32),
            pltpu.VMEM((K, D), jnp.float32),
            pltpu.VMEM((K,), jnp.float32),
            pltpu.VMEM_SHARED((N_NODES,), jnp.float32),
            pltpu.VMEM_SHARED((N_NODES,), jnp.float32),
            pltpu.VMEM_SHARED((N_NODES, D), jnp.float32),
            pltpu.SemaphoreType.DMA((NSLOT,)),
            pltpu.SemaphoreType.DMA((NSLOT,)),
            pltpu.SemaphoreType.DMA((2,)),
            pltpu.SemaphoreType.DMA((2,)),
            pltpu.SemaphoreType.DMA((2,)),
            pltpu.SemaphoreType.DMA((2,)),
        ],
        compiler_params=pltpu.CompilerParams(needs_layout_passes=False),
    )
    return kfn(fj, a1, a2, src_r, dst_r)


def _combine_body(p0_ref, p1_ref, base_ref, out_ref):
    out_ref[:] = p0_ref[:] + p1_ref[:] + base_ref[:]


def _combine_stage(p0, p1, base):
    R = 1000
    return pl.pallas_call(
        _combine_body,
        grid=(N_NODES // R,),
        in_specs=[pl.BlockSpec((R, D), lambda i: (i, 0))] * 3,
        out_specs=pl.BlockSpec((R, D), lambda i: (i, 0)),
        out_shape=jax.ShapeDtypeStruct((N_NODES, D), jnp.float32),
    )(p0, p1, base)


def kernel(features, adj_indices, W1, b1, W2, b2, a1_w, a1_b, a2_w, a2_b):
    adj = adj_indices.astype(jnp.int32)
    src_r = adj[0]
    dst_r = adj[1]

    fj, base, a1, a2 = _dense_stage(
        features,
        W1.T, b1.reshape(1, D),
        W2.T, b2.reshape(1, D),
        a1_w.reshape(1, D).T, a1_b.reshape(1, 1),
        a2_w.reshape(1, D).T, a2_b.reshape(1, 1),
    )

    part = _edge_stage(fj, a1.reshape(N_NODES), a2.reshape(N_NODES),
                       src_r, dst_r)

    return _combine_stage(part[0], part[1], base)


# speedup vs baseline: 1.0929x; 1.0137x over previous
import functools

import jax
import jax.numpy as jnp
from jax import lax
from jax.experimental import pallas as pl
from jax.experimental.pallas import tpu as pltpu
from jax.experimental.pallas import tpu_sc as plsc

N_NODES = 10000
N_EDGES = 320000
D = 128

NC = 2
NS = 16
NW = NC * NS
E_PER_W = N_EDGES // NW
K = 80
NCH = E_PER_W // K
RPT = 632
PIECE = 80


def _dense_body(x_ref, w1t_ref, b1_ref, w2t_ref, b2_ref, a1wt_ref, a1b_ref,
                a2wt_ref, a2b_ref, fj_ref, base_ref, a1_ref, a2_ref):
    x = x_ref[:]
    fi = jnp.maximum(
        jnp.dot(x, w1t_ref[:], preferred_element_type=jnp.float32) + b1_ref[:], 0.0)
    fj = jnp.maximum(
        jnp.dot(x, w2t_ref[:], preferred_element_type=jnp.float32) + b2_ref[:], 0.0)
    a1 = jnp.dot(fi, a1wt_ref[:], preferred_element_type=jnp.float32) + a1b_ref[0, 0]
    a2 = jnp.dot(fj, a2wt_ref[:], preferred_element_type=jnp.float32) + a2b_ref[0, 0]
    att_self = jax.nn.sigmoid(a1 + a2)
    fj_ref[:] = fj
    base_ref[:] = fi + att_self * fj
    a1_ref[:] = a1
    a2_ref[:] = a2


def _dense_stage(features, W1T, b1, W2T, b2, a1wT, a1b, a2wT, a2b):
    R = 1000
    grid = N_NODES // R
    full = lambda shp: pl.BlockSpec(shp, lambda i: (0, 0))
    return pl.pallas_call(
        _dense_body,
        grid=(grid,),
        in_specs=[
            pl.BlockSpec((R, D), lambda i: (i, 0)),
            full((D, D)), full((1, D)), full((D, D)), full((1, D)),
            full((D, 1)), full((1, 1)), full((D, 1)), full((1, 1)),
        ],
        out_specs=[
            pl.BlockSpec((R, D), lambda i: (i, 0)),
            pl.BlockSpec((R, D), lambda i: (i, 0)),
            pl.BlockSpec((R, 1), lambda i: (i, 0)),
            pl.BlockSpec((R, 1), lambda i: (i, 0)),
        ],
        out_shape=[
            jax.ShapeDtypeStruct((N_NODES, D), jnp.float32),
            jax.ShapeDtypeStruct((N_NODES, D), jnp.float32),
            jax.ShapeDtypeStruct((N_NODES, 1), jnp.float32),
            jax.ShapeDtypeStruct((N_NODES, 1), jnp.float32),
        ],
    )(features, W1T, b1, W2T, b2, a1wT, a1b, a2wT, a2b)


NSLOT = 5


def _edge_body(fj_hbm, a1_hbm, a2_hbm, src_hbm, dst_hbm, part_hbm,
               src_c, dst_c, a1c, a2c, raw0, raw1, scd0, scd1, att_v,
               a1_sh, a2_sh, acc,
               sem_src, sem_dst, sem_g, sem_a1, sem_a2, sem_sc):
    c = lax.axis_index("c")
    s = lax.axis_index("s")
    wid = c * NS + s
    ebase = wid * E_PER_W
    base = s * RPT
    rows_total = jnp.where(s < NS - 1, RPT, N_NODES - (NS - 1) * RPT)
    n_pieces = (rows_total + PIECE - 1) // PIECE

    def _zrow(i, carry):
        z = jnp.zeros((16,), jnp.float32)
        for q in range(8):
            raw0[i, pl.ds(q * 16, 16)] = z
        return carry
    lax.fori_loop(0, PIECE, _zrow, 0)

    def _zpiece(k, carry):
        off = base + jnp.minimum(k * PIECE, rows_total - PIECE)
        pltpu.async_copy(raw0, acc.at[pl.ds(off, PIECE)], sem_g.at[0])
        return carry
    lax.fori_loop(0, n_pieces, _zpiece, 0)

    def _zdrain(k, carry):
        off = base + jnp.minimum(k * PIECE, rows_total - PIECE)
        pltpu.make_async_copy(raw0, acc.at[pl.ds(off, PIECE)],
                              sem_g.at[0]).wait()
        return carry
    lax.fori_loop(0, n_pieces, _zdrain, 0)

    @pl.when(s == 0)
    def _():
        pltpu.sync_copy(a1_hbm, a1_sh)

    @pl.when(s == 1)
    def _():
        pltpu.sync_copy(a2_hbm, a2_sh)

    plsc.subcore_barrier()

    raws = (raw0, raw1)
    scds = (scd0, scd1)

    def idx_start(j):
        m = j % NSLOT
        pltpu.async_copy(src_hbm.at[pl.ds(ebase + j * K, K)], src_c.at[m],
                         sem_src.at[m])
        pltpu.async_copy(dst_hbm.at[pl.ds(ebase + j * K, K)], dst_c.at[m],
                         sem_dst.at[m])

    def fetch_start(j, b):
        m = j % NSLOT
        pltpu.make_async_copy(src_hbm.at[pl.ds(ebase + j * K, K)],
                              src_c.at[m], sem_src.at[m]).wait()
        pltpu.make_async_copy(dst_hbm.at[pl.ds(ebase + j * K, K)],
                              dst_c.at[m], sem_dst.at[m]).wait()
        pltpu.async_copy(fj_hbm.at[dst_c.at[m]], raws[b], sem_g.at[b])
        pltpu.async_copy(a1_sh.at[src_c.at[m]], a1c.at[b], sem_a1.at[b])
        pltpu.async_copy(a2_sh.at[dst_c.at[m]], a2c.at[b], sem_a2.at[b])

    def proc(j, b, wait_prev):
        m = j % NSLOT
        raw = raws[b]
        scd = scds[b]
        pltpu.make_async_copy(a1_sh.at[src_c.at[m]], a1c.at[b],
                              sem_a1.at[b]).wait()
        pltpu.make_async_copy(a2_sh.at[dst_c.at[m]], a2c.at[b],
                              sem_a2.at[b]).wait()

        for g in range(5):
            x = a1c[b, pl.ds(g * 16, 16)] + a2c[b, pl.ds(g * 16, 16)]
            att_v[pl.ds(g * 16, 16)] = 1.0 / (1.0 + jnp.exp(-x))

        pltpu.make_async_copy(fj_hbm.at[dst_c.at[m]], raw, sem_g.at[b]).wait()
        if wait_prev:
            m2 = (j - 2) % NSLOT
            pltpu.make_async_copy(scd, acc.at[src_c.at[m2]],
                                  sem_sc.at[b]).wait()

        @plsc.parallel_loop(0, K, 1, unroll=4)
        def _scale(e):
            idx = lax.broadcast_in_dim(e, (16,), ())
            a = plsc.load_gather(att_v, [idx])
            for q in range(8):
                scd[e, pl.ds(q * 16, 16)] = raw[e, pl.ds(q * 16, 16)] * a

        pltpu.async_copy(scd, acc.at[src_c.at[m]], sem_sc.at[b], add=True)

    idx_start(0)
    idx_start(1)
    idx_start(2)
    idx_start(3)
    fetch_start(0, 0)
    fetch_start(1, 1)
    proc(0, 0, False)
    fetch_start(2, 0)
    proc(1, 1, False)

    def _pair(t, carry):
        ja = 2 * t
        jb = ja + 1
        idx_start(ja + 2)
        fetch_start(jb, 1)
        proc(ja, 0, True)
        idx_start(jb + 2)
        fetch_start(jb + 1, 0)
        proc(jb, 1, True)
        return carry
    lax.fori_loop(1, 61, _pair, 0)

    idx_start(124)
    fetch_start(123, 1)
    proc(122, 0, True)
    fetch_start(124, 0)
    proc(123, 1, True)
    proc(124, 0, True)
    pltpu.make_async_copy(scds[1], acc.at[src_c.at[123 % NSLOT]],
                          sem_sc.at[1]).wait()
    pltpu.make_async_copy(scds[0], acc.at[src_c.at[124 % NSLOT]],
                          sem_sc.at[0]).wait()

    plsc.subcore_barrier()

    def _cpiece(k, carry):
        off = base + jnp.minimum(k * PIECE, rows_total - PIECE)
        pltpu.sync_copy(acc.at[pl.ds(off, PIECE)], raw0)
        pltpu.sync_copy(raw0, part_hbm.at[c].at[pl.ds(off, PIECE)])
        return carry
    lax.fori_loop(0, n_pieces, _cpiece, 0)


def _edge_stage(fj, a1, a2, src_r, dst_r):
    mesh = plsc.VectorSubcoreMesh(
        core_axis_name="c", subcore_axis_name="s", num_cores=NC, num_subcores=NS)
    kfn = pl.kernel(
        _edge_body,
        out_type=jax.ShapeDtypeStruct((NC, N_NODES, D), jnp.float32),
        mesh=mesh,
        scratch_types=[
            pltpu.VMEM((NSLOT, K), jnp.int32),
            pltpu.VMEM((NSLOT, K), jnp.int32),
            pltpu.VMEM((2, K), jnp.float32),
            pltpu.VMEM((2, K), jnp.float32),
            pltpu.VMEM((K, D), jnp.float32),
            pltpu.VMEM((K, D), jnp.float32),
            pltpu.VMEM((K, D), jnp.float32),
            pltpu.VMEM((K, D), jnp.float32),
            pltpu.VMEM((K,), jnp.float32),
            pltpu.VMEM_SHARED((N_NODES,), jnp.float32),
            pltpu.VMEM_SHARED((N_NODES,), jnp.float32),
            pltpu.VMEM_SHARED((N_NODES, D), jnp.float32),
            pltpu.SemaphoreType.DMA((NSLOT,)),
            pltpu.SemaphoreType.DMA((NSLOT,)),
            pltpu.SemaphoreType.DMA((2,)),
            pltpu.SemaphoreType.DMA((2,)),
            pltpu.SemaphoreType.DMA((2,)),
            pltpu.SemaphoreType.DMA((2,)),
        ],
        compiler_params=pltpu.CompilerParams(needs_layout_passes=False),
    )
    return kfn(fj, a1, a2, src_r, dst_r)


def _combine_body(p0_ref, p1_ref, base_ref, out_ref):
    out_ref[:] = p0_ref[:] + p1_ref[:] + base_ref[:]


def _combine_stage(p0, p1, base):
    R = 1000
    return pl.pallas_call(
        _combine_body,
        grid=(N_NODES // R,),
        in_specs=[pl.BlockSpec((R, D), lambda i: (i, 0))] * 3,
        out_specs=pl.BlockSpec((R, D), lambda i: (i, 0)),
        out_shape=jax.ShapeDtypeStruct((N_NODES, D), jnp.float32),
    )(p0, p1, base)


def kernel(features, adj_indices, W1, b1, W2, b2, a1_w, a1_b, a2_w, a2_b):
    adj = adj_indices.astype(jnp.int32)
    src_r = adj[0]
    dst_r = adj[1]

    fj, base, a1, a2 = _dense_stage(
        features,
        W1.T, b1.reshape(1, D),
        W2.T, b2.reshape(1, D),
        a1_w.reshape(1, D).T, a1_b.reshape(1, 1),
        a2_w.reshape(1, D).T, a2_b.reshape(1, 1),
    )

    part = _edge_stage(fj, a1.reshape(N_NODES), a2.reshape(N_NODES),
                       src_r, dst_r)

    return _combine_stage(part[0], part[1], base)


# speedup vs baseline: 1.0978x; 1.0044x over previous
import functools

import jax
import jax.numpy as jnp
from jax import lax
from jax.experimental import pallas as pl
from jax.experimental.pallas import tpu as pltpu
from jax.experimental.pallas import tpu_sc as plsc

N_NODES = 10000
N_EDGES = 320000
D = 128

NC = 2
NS = 16
NW = NC * NS
E_PER_W = N_EDGES // NW
K = 80
NCH = E_PER_W // K
RPT = 632
PIECE = 80


def _dense_body(x_ref, w1t_ref, b1_ref, w2t_ref, b2_ref, a1wt_ref, a1b_ref,
                a2wt_ref, a2b_ref, fj_ref, base_ref, a1_ref, a2_ref):
    x = x_ref[:]
    fi = jnp.maximum(
        jnp.dot(x, w1t_ref[:], preferred_element_type=jnp.float32) + b1_ref[:], 0.0)
    fj = jnp.maximum(
        jnp.dot(x, w2t_ref[:], preferred_element_type=jnp.float32) + b2_ref[:], 0.0)
    a1 = jnp.dot(fi, a1wt_ref[:], preferred_element_type=jnp.float32) + a1b_ref[0, 0]
    a2 = jnp.dot(fj, a2wt_ref[:], preferred_element_type=jnp.float32) + a2b_ref[0, 0]
    att_self = jax.nn.sigmoid(a1 + a2)
    fj_ref[:] = fj
    base_ref[:] = fi + att_self * fj
    a1_ref[:] = a1
    a2_ref[:] = a2


def _dense_stage(features, W1T, b1, W2T, b2, a1wT, a1b, a2wT, a2b):
    R = 1000
    grid = N_NODES // R
    full = lambda shp: pl.BlockSpec(shp, lambda i: (0, 0))
    return pl.pallas_call(
        _dense_body,
        grid=(grid,),
        in_specs=[
            pl.BlockSpec((R, D), lambda i: (i, 0)),
            full((D, D)), full((1, D)), full((D, D)), full((1, D)),
            full((D, 1)), full((1, 1)), full((D, 1)), full((1, 1)),
        ],
        out_specs=[
            pl.BlockSpec((R, D), lambda i: (i, 0)),
            pl.BlockSpec((R, D), lambda i: (i, 0)),
            pl.BlockSpec((R, 1), lambda i: (i, 0)),
            pl.BlockSpec((R, 1), lambda i: (i, 0)),
        ],
        out_shape=[
            jax.ShapeDtypeStruct((N_NODES, D), jnp.float32),
            jax.ShapeDtypeStruct((N_NODES, D), jnp.float32),
            jax.ShapeDtypeStruct((N_NODES, 1), jnp.float32),
            jax.ShapeDtypeStruct((N_NODES, 1), jnp.float32),
        ],
    )(features, W1T, b1, W2T, b2, a1wT, a1b, a2wT, a2b)


NSLOT = 5


def _edge_body(fj_hbm, a1_hbm, a2_hbm, src_hbm, dst_hbm, part_hbm,
               src_c, dst_c, a1c, a2c, raw0, raw1, scd0, scd1, att_v,
               a1_sh, a2_sh, acc,
               sem_src, sem_dst, sem_g, sem_a1, sem_a2, sem_sc):
    c = lax.axis_index("c")
    s = lax.axis_index("s")
    wid = c * NS + s
    ebase = wid * E_PER_W
    base = s * RPT
    rows_total = jnp.where(s < NS - 1, RPT, N_NODES - (NS - 1) * RPT)
    n_pieces = (rows_total + PIECE - 1) // PIECE

    def _zrow(i, carry):
        z = jnp.zeros((16,), jnp.float32)
        for q in range(8):
            raw0[i, pl.ds(q * 16, 16)] = z
        return carry
    lax.fori_loop(0, PIECE, _zrow, 0)

    def _zpiece(k, carry):
        off = base + jnp.minimum(k * PIECE, rows_total - PIECE)
        pltpu.async_copy(raw0, acc.at[pl.ds(off, PIECE)], sem_g.at[0])
        return carry
    lax.fori_loop(0, n_pieces, _zpiece, 0)

    def _zdrain(k, carry):
        off = base + jnp.minimum(k * PIECE, rows_total - PIECE)
        pltpu.make_async_copy(raw0, acc.at[pl.ds(off, PIECE)],
                              sem_g.at[0]).wait()
        return carry
    lax.fori_loop(0, n_pieces, _zdrain, 0)

    @pl.when(s == 0)
    def _():
        pltpu.sync_copy(a1_hbm, a1_sh)

    @pl.when(s == 1)
    def _():
        pltpu.sync_copy(a2_hbm, a2_sh)

    plsc.subcore_barrier()

    raws = (raw0, raw1)
    scds = (scd0, scd1)

    def idx_start(j):
        m = j % NSLOT
        pltpu.async_copy(src_hbm.at[pl.ds(ebase + j * K, K)], src_c.at[m],
                         sem_src.at[m])
        pltpu.async_copy(dst_hbm.at[pl.ds(ebase + j * K, K)], dst_c.at[m],
                         sem_dst.at[m])

    def fetch_start(j, b):
        m = j % NSLOT
        pltpu.make_async_copy(src_hbm.at[pl.ds(ebase + j * K, K)],
                              src_c.at[m], sem_src.at[m]).wait()
        pltpu.make_async_copy(dst_hbm.at[pl.ds(ebase + j * K, K)],
                              dst_c.at[m], sem_dst.at[m]).wait()
        pltpu.async_copy(fj_hbm.at[dst_c.at[m]], raws[b], sem_g.at[b])
        pltpu.async_copy(a1_sh.at[src_c.at[m]], a1c.at[b], sem_a1.at[b])
        pltpu.async_copy(a2_sh.at[dst_c.at[m]], a2c.at[b], sem_a2.at[b])

    def proc(j, b, wait_prev):
        m = j % NSLOT
        raw = raws[b]
        scd = scds[b]
        pltpu.make_async_copy(a1_sh.at[src_c.at[m]], a1c.at[b],
                              sem_a1.at[b]).wait()
        pltpu.make_async_copy(a2_sh.at[dst_c.at[m]], a2c.at[b],
                              sem_a2.at[b]).wait()

        for g in range(5):
            x = a1c[b, pl.ds(g * 16, 16)] + a2c[b, pl.ds(g * 16, 16)]
            att_v[pl.ds(g * 16, 16)] = 1.0 / (1.0 + jnp.exp(-x))

        pltpu.make_async_copy(fj_hbm.at[dst_c.at[m]], raw, sem_g.at[b]).wait()
        if wait_prev:
            m2 = (j - 2) % NSLOT
            pltpu.make_async_copy(scd, acc.at[src_c.at[m2]],
                                  sem_sc.at[b]).wait()

        @plsc.parallel_loop(0, K, 1, unroll=4)
        def _scale(e):
            idx = lax.broadcast_in_dim(e, (16,), ())
            a = plsc.load_gather(att_v, [idx])
            for q in range(8):
                scd[e, pl.ds(q * 16, 16)] = raw[e, pl.ds(q * 16, 16)] * a

        pltpu.async_copy(scd, acc.at[src_c.at[m]], sem_sc.at[b], add=True)

    idx_start(0)
    idx_start(1)
    idx_start(2)
    idx_start(3)
    fetch_start(0, 0)
    fetch_start(1, 1)
    proc(0, 0, False)
    fetch_start(2, 0)
    proc(1, 1, False)

    def _pair(t, carry):
        ja = 2 * t
        jb = ja + 1
        idx_start(ja + 2)
        fetch_start(jb, 1)
        proc(ja, 0, True)
        idx_start(jb + 2)
        fetch_start(jb + 1, 0)
        proc(jb, 1, True)
        return carry
    lax.fori_loop(1, 61, _pair, 0)

    idx_start(124)
    fetch_start(123, 1)
    proc(122, 0, True)
    fetch_start(124, 0)
    proc(123, 1, True)
    proc(124, 0, True)
    pltpu.make_async_copy(scds[1], acc.at[src_c.at[123 % NSLOT]],
                          sem_sc.at[1]).wait()
    pltpu.make_async_copy(scds[0], acc.at[src_c.at[124 % NSLOT]],
                          sem_sc.at[0]).wait()

    plsc.subcore_barrier()

    def _cpiece(k, carry):
        off = base + jnp.minimum(k * PIECE, rows_total - PIECE)
        pltpu.async_copy(acc.at[pl.ds(off, PIECE)],
                         part_hbm.at[c].at[pl.ds(off, PIECE)], sem_g.at[0])
        return carry
    lax.fori_loop(0, n_pieces, _cpiece, 0)

    def _cdrain(k, carry):
        off = base + jnp.minimum(k * PIECE, rows_total - PIECE)
        pltpu.make_async_copy(acc.at[pl.ds(off, PIECE)],
                              part_hbm.at[c].at[pl.ds(off, PIECE)],
                              sem_g.at[0]).wait()
        return carry
    lax.fori_loop(0, n_pieces, _cdrain, 0)


def _edge_stage(fj, a1, a2, src_r, dst_r):
    mesh = plsc.VectorSubcoreMesh(
        core_axis_name="c", subcore_axis_name="s", num_cores=NC, num_subcores=NS)
    kfn = pl.kernel(
        _edge_body,
        out_type=jax.ShapeDtypeStruct((NC, N_NODES, D), jnp.float32),
        mesh=mesh,
        scratch_types=[
            pltpu.VMEM((NSLOT, K), jnp.int32),
            pltpu.VMEM((NSLOT, K), jnp.int32),
            pltpu.VMEM((2, K), jnp.float32),
            pltpu.VMEM((2, K), jnp.float32),
            pltpu.VMEM((K, D), jnp.float32),
            pltpu.VMEM((K, D), jnp.float32),
            pltpu.VMEM((K, D), jnp.float32),
            pltpu.VMEM((K, D), jnp.float32),
            pltpu.VMEM((K,), jnp.float32),
            pltpu.VMEM_SHARED((N_NODES,), jnp.float32),
            pltpu.VMEM_SHARED((N_NODES,), jnp.float32),
            pltpu.VMEM_SHARED((N_NODES, D), jnp.float32),
            pltpu.SemaphoreType.DMA((NSLOT,)),
            pltpu.SemaphoreType.DMA((NSLOT,)),
            pltpu.SemaphoreType.DMA((2,)),
            pltpu.SemaphoreType.DMA((2,)),
            pltpu.SemaphoreType.DMA((2,)),
            pltpu.SemaphoreType.DMA((2,)),
        ],
        compiler_params=pltpu.CompilerParams(needs_layout_passes=False),
    )
    return kfn(fj, a1, a2, src_r, dst_r)


def _combine_body(p0_ref, p1_ref, base_ref, out_ref):
    out_ref[:] = p0_ref[:] + p1_ref[:] + base_ref[:]


def _combine_stage(p0, p1, base):
    R = 1000
    return pl.pallas_call(
        _combine_body,
        grid=(N_NODES // R,),
        in_specs=[pl.BlockSpec((R, D), lambda i: (i, 0))] * 3,
        out_specs=pl.BlockSpec((R, D), lambda i: (i, 0)),
        out_shape=jax.ShapeDtypeStruct((N_NODES, D), jnp.float32),
    )(p0, p1, base)


def kernel(features, adj_indices, W1, b1, W2, b2, a1_w, a1_b, a2_w, a2_b):
    adj = adj_indices.astype(jnp.int32)
    src_r = adj[0]
    dst_r = adj[1]

    fj, base, a1, a2 = _dense_stage(
        features,
        W1.T, b1.reshape(1, D),
        W2.T, b2.reshape(1, D),
        a1_w.reshape(1, D).T, a1_b.reshape(1, 1),
        a2_w.reshape(1, D).T, a2_b.reshape(1, 1),
    )

    part = _edge_stage(fj, a1.reshape(N_NODES), a2.reshape(N_NODES),
                       src_r, dst_r)

    return _combine_stage(part[0], part[1], base)
